# baseline (speedup 1.0000x reference)
import functools

import jax
import jax.numpy as jnp
from jax import lax
from jax.experimental import pallas as pl
from jax.experimental.pallas import tpu as pltpu
from jax.experimental.pallas import tpu_sc as plsc

_EPS = 1e-5
_B = 16384
_D = 64
_F = 2 * _D
_NC = 2
_NS = 16
_NW = _NC * _NS
_BPW = _B // _NW
_CH = 256


def _make_sc_gather():
    mesh = plsc.VectorSubcoreMesh(core_axis_name="c", subcore_axis_name="s")
    out_t = tuple(
        jax.ShapeDtypeStruct((_B, _D), jnp.float32) for _ in range(4)
    )

    @functools.partial(
        pl.kernel,
        mesh=mesh,
        out_type=out_t,
        scratch_types=[
            pltpu.VMEM((_BPW,), jnp.int32),
            pltpu.VMEM((_BPW,), jnp.int32),
            pltpu.SemaphoreType.DMA,
            pltpu.SemaphoreType.DMA,
            pltpu.SemaphoreType.DMA,
            pltpu.SemaphoreType.DMA,
        ],
    )
    def sc_gather(ug_hbm, um_hbm, ig_hbm, im_hbm, uid_hbm, iid_hbm,
                  oug, oum, oig, oim,
                  us, its, sem0, sem1, sem2, sem3):
        wid = lax.axis_index("s") * _NC + lax.axis_index("c")
        base = wid * _BPW
        pltpu.sync_copy(uid_hbm.at[pl.ds(base, _BPW)], us)
        pltpu.sync_copy(iid_hbm.at[pl.ds(base, _BPW)], its)

        work = [
            (ug_hbm, us, oug, sem0),
            (um_hbm, us, oum, sem1),
            (ig_hbm, its, oig, sem2),
            (im_hbm, its, oim, sem3),
        ]
        for tbl, idx_s, out, sem in work:
            def body(g, _, tbl=tbl, idx_s=idx_s, out=out, sem=sem):
                v = idx_s[pl.ds(g * 16, 16)]
                for l in range(16):
                    pltpu.async_copy(
                        tbl.at[pl.ds(v[l], 1)],
                        out.at[pl.ds(base + g * 16 + l, 1)],
                        sem)
                return 0

            lax.fori_loop(0, _BPW // 16, body, 0)
        for tbl, idx_s, out, sem in work:
            pltpu.make_async_copy(
                tbl.at[pl.ds(0, _BPW)],
                out.at[pl.ds(base, _BPW)],
                sem).wait()

    return sc_gather


_SC_GATHER_CACHE = []


def _sc_gather(*args):
    if not _SC_GATHER_CACHE:
        _SC_GATHER_CACHE.append(_make_sc_gather())
    return _SC_GATHER_CACHE[0](*args)


_BLK = 4096


def _tc_body(ug_r, um_r, ig_r, im_r,
             w1u_r, w1i_r, b1_r, s1_r, e1_r,
             w2_r, b2_r, s2_r, e2_r,
             w3_r, b3_r, s3_r, e3_r,
             wfg_r, wfh_r, bf_r, out_r):
    um = um_r[...]
    im = im_r[...]
    h = jnp.dot(um, w1u_r[...], preferred_element_type=jnp.float32)
    h += jnp.dot(im, w1i_r[...], preferred_element_type=jnp.float32)
    h = jnp.maximum(h + b1_r[...], 0.0) * s1_r[...] + e1_r[...]
    h = jnp.dot(h, w2_r[...], preferred_element_type=jnp.float32)
    h = jnp.maximum(h + b2_r[...], 0.0) * s2_r[...] + e2_r[...]
    h = jnp.dot(h, w3_r[...], preferred_element_type=jnp.float32)
    h = jnp.maximum(h + b3_r[...], 0.0) * s3_r[...] + e3_r[...]
    g = ug_r[...] * ig_r[...]
    logit = jnp.sum(g * wfg_r[...] + h * wfh_r[...], axis=1) + bf_r[0, 0]
    out_r[...] = jax.nn.sigmoid(logit)


def _tc_forward(rug, rum, rig, rim, w1u, w1i, b1, s1, e1,
                w2, b2, s2, e2, w3, b3, s3, e3, wfg, wfh, bf):
    n_blk = _B // _BLK
    row_spec = pl.BlockSpec((_BLK, _D), lambda i: (i, 0))

    def full(shape):
        return pl.BlockSpec(shape, lambda i: tuple(0 for _ in shape))

    return pl.pallas_call(
        _tc_body,
        grid=(n_blk,),
        in_specs=[
            row_spec, row_spec, row_spec, row_spec,
            full((_D, 256)), full((_D, 256)), full((1, 256)), full((1, 256)),
            full((1, 256)),
            full((256, 128)), full((1, 128)), full((1, 128)), full((1, 128)),
            full((128, 64)), full((1, 64)), full((1, 64)), full((1, 64)),
            full((1, _D)), full((1, _D)), full((1, 1)),
        ],
        out_specs=pl.BlockSpec((_BLK,), lambda i: (i,)),
        out_shape=jax.ShapeDtypeStruct((_B,), jnp.float32),
    )(rug, rum, rig, rim, w1u, w1i, b1, s1, e1,
      w2, b2, s2, e2, w3, b3, s3, e3, wfg, wfh, bf)


def kernel(user_ids, item_ids, user_gmf, item_gmf, user_mlp, item_mlp,
           W1, b1, g1, be1, W2, b2, g2, be2, W3, b3, g3, be3, Wf, bf):
    rug, rum, rig, rim = _sc_gather(
        user_gmf, user_mlp, item_gmf, item_mlp, user_ids, item_ids)

    inv = 1.0 / jnp.sqrt(jnp.float32(1.0) + jnp.float32(_EPS))
    s1 = (g1 * inv).reshape(1, -1)
    s2 = (g2 * inv).reshape(1, -1)
    s3 = (g3 * inv).reshape(1, -1)
    return _tc_forward(
        rug, rum, rig, rim,
        W1[:_D], W1[_D:], b1.reshape(1, -1), s1, be1.reshape(1, -1),
        W2, b2.reshape(1, -1), s2, be2.reshape(1, -1),
        W3, b3.reshape(1, -1), s3, be3.reshape(1, -1),
        Wf[:_D, 0].reshape(1, -1), Wf[_D:, 0].reshape(1, -1),
        bf.reshape(1, 1),
    )
